# baseline (speedup 1.0000x reference)
import jax
import jax.numpy as jnp
from jax import lax
from jax.experimental import pallas as pl
from jax.experimental.pallas import tpu as pltpu
from jax.experimental.pallas import tpu_sc as plsc

CTX = 2048
DIM = 64
BATCH = 4

NC = 2
NS = 16
NW = NC * NS
CBLK = 128
SC_CB = 8
TC_CB = CTX // CBLK - SC_CB
SC_CTX = SC_CB * CBLK
LANES = 16
NBUF = 8
TOK = CBLK
NGRP = TOK // NBUF

TC_NBUF = 16
TC_UNITS = BATCH * TC_CB
TC_TOK = TC_UNITS * CBLK


def _sc_body(x_hbm, tokT_hbm, posT_hbm, out_hbm, idx_v, tcol_v, outT_v,
             idx_s, *sems):
    c = lax.axis_index("c")
    s = lax.axis_index("s")
    wid = s * NC + c
    cb = wid % SC_CB
    bb = wid // SC_CB
    c0 = cb * CBLK

    pltpu.sync_copy(
        x_hbm.at[pl.ds(bb, 1), pl.ds(c0, CBLK)], idx_v.at[pl.ds(0, 1)]
    )

    pltpu.sync_copy(posT_hbm.at[:, pl.ds(c0, CBLK)], outT_v)

    for k in range(TOK // LANES):
        vec = idx_v[0, pl.ds(k * LANES, LANES)]
        for i in range(LANES):
            idx_s[k * LANES + i] = vec[i]

    iotas = [lax.iota(jnp.int32, LANES) + kk * LANES for kk in range(DIM // LANES)]

    def fire(t, b):
        v = idx_s[t]
        off = pl.multiple_of((v >> 7) * 128, 128)
        pltpu.async_copy(tokT_hbm.at[:, pl.ds(off, 128)], tcol_v.at[b], sems[b])

    def process(t, b):
        v = idx_s[t]
        vl = jnp.full((LANES,), v & 127, jnp.int32)
        bsp = jnp.full((LANES,), b, jnp.int32)
        tsp = jnp.full((LANES,), t, jnp.int32)
        for kk in range(DIM // LANES):
            col = plsc.load_gather(tcol_v, [bsp, iotas[kk], vl])
            plsc.addupdate_scatter(outT_v, [iotas[kk], tsp], col)

    def drain(b):
        pltpu.make_async_copy(
            tokT_hbm.at[:, pl.ds(0, 128)], tcol_v.at[b], sems[b]
        ).wait()

    for b in range(NBUF):
        fire(b, b)

    def group(g, carry):
        for b in range(NBUF):
            t = g * NBUF + b
            drain(b)
            process(t, b)
            fire(t + NBUF, b)
        return carry

    lax.fori_loop(0, NGRP - 1, group, 0)

    for b in range(NBUF):
        t = (NGRP - 1) * NBUF + b
        drain(b)
        process(t, b)

    pltpu.sync_copy(outT_v, out_hbm.at[bb, :, pl.ds(c0, CBLK)])


def _tc_body(sidx_ref, tokT_ref, pos_ref, out_ref, buf, sems):
    i = pl.program_id(0)
    t0 = i * CBLK

    def fire(t, b):
        v = sidx_ref[t]
        off = pl.multiple_of((v >> 7) * 128, 128)
        pltpu.make_async_copy(
            tokT_ref.at[:, pl.ds(off, 128)], buf.at[b], sems.at[b]
        ).start()

    def drain(b):
        pltpu.make_async_copy(
            tokT_ref.at[:, pl.ds(0, 128)], buf.at[b], sems.at[b]
        ).wait()

    @pl.when(i == 0)
    def _():
        for b in range(TC_NBUF):
            fire(b, b)

    lane = lax.broadcasted_iota(jnp.int32, (128, 1), 0)
    for j in range(CBLK):
        b = j % TC_NBUF
        t = t0 + j
        v = sidx_ref[t]
        drain(b)
        e = (lane == (v & 127)).astype(jnp.float32)
        col = jax.lax.dot(buf[b], e, preferred_element_type=jnp.float32)
        out_ref[0, :, pl.ds(j, 1)] = col + pos_ref[:, pl.ds(j, 1)]

        @pl.when(t + TC_NBUF < TC_TOK)
        def _():
            fire(t + TC_NBUF, b)


@jax.jit
def kernel(x, token_table, pos_table):
    tokT = token_table.T
    posT = pos_table.T
    mesh = plsc.VectorSubcoreMesh(core_axis_name="c", subcore_axis_name="s")
    sc_out = pl.kernel(
        _sc_body,
        out_type=jax.ShapeDtypeStruct((BATCH, DIM, SC_CTX), jnp.float32),
        mesh=mesh,
        scratch_types=[
            pltpu.VMEM((1, CBLK), jnp.int32),
            pltpu.VMEM((NBUF, DIM, 128), jnp.float32),
            pltpu.VMEM((DIM, TOK), jnp.float32),
            pltpu.SMEM((TOK,), jnp.int32),
        ] + [pltpu.SemaphoreType.DMA] * NBUF,
        compiler_params=pltpu.CompilerParams(needs_layout_passes=False),
    )(x, tokT, posT)

    x_tc = x[:, SC_CTX:].reshape(-1).astype(jnp.int32)
    grid_spec = pltpu.PrefetchScalarGridSpec(
        num_scalar_prefetch=1,
        grid=(TC_UNITS,),
        in_specs=[
            pl.BlockSpec(memory_space=pltpu.HBM),
            pl.BlockSpec(
                (DIM, CBLK), lambda i, sref: (0, SC_CB + (i % TC_CB))
            ),
        ],
        out_specs=pl.BlockSpec(
            (1, DIM, CBLK), lambda i, sref: (i // TC_CB, 0, i % TC_CB)
        ),
        scratch_shapes=[
            pltpu.VMEM((TC_NBUF, DIM, 128), jnp.float32),
            pltpu.SemaphoreType.DMA((TC_NBUF,)),
        ],
    )
    tc_out = pl.pallas_call(
        _tc_body,
        grid_spec=grid_spec,
        out_shape=jax.ShapeDtypeStruct((BATCH, DIM, CTX - SC_CTX), jnp.float32),
    )(x_tc, tokT, posT)

    out = jnp.concatenate([sc_out, tc_out], axis=2)
    return jnp.transpose(out, (0, 2, 1))


# speedup vs baseline: 7.1566x; 7.1566x over previous
import jax
import jax.numpy as jnp
from jax import lax
from jax.experimental import pallas as pl
from jax.experimental.pallas import tpu as pltpu
from jax.experimental.pallas import tpu_sc as plsc

CTX = 2048
DIM = 64
BATCH = 4

NC = 2
NS = 16
NW = NC * NS
CBLK = 128
NB_PER_W = 2
TOK = NB_PER_W * CBLK
LANES = 16
NBUF = 12
NGRP = TOK // NBUF
REM = TOK - NGRP * NBUF


def _emb_body(x_hbm, tokT_hbm, posT_hbm, out_hbm, idx_v, tcol_v, outT_v,
              idx_s, *sems):
    c = lax.axis_index("c")
    s = lax.axis_index("s")
    wid = s * NC + c
    cb = wid % (CTX // CBLK)
    bb = wid // (CTX // CBLK)
    c0 = cb * CBLK

    for i in range(NB_PER_W):
        pltpu.sync_copy(
            x_hbm.at[pl.ds(NB_PER_W * bb + i, 1), pl.ds(c0, CBLK)],
            idx_v.at[pl.ds(i, 1)],
        )

    for i in range(NB_PER_W):
        pltpu.sync_copy(
            posT_hbm.at[:, pl.ds(c0, CBLK)],
            outT_v.at[:, pl.ds(i * CBLK, CBLK)],
        )

    for k in range(TOK // LANES):
        vec = idx_v[k // (CBLK // LANES), pl.ds((k % (CBLK // LANES)) * LANES, LANES)]
        for i in range(LANES):
            idx_s[k * LANES + i] = vec[i]

    iotas = [lax.iota(jnp.int32, LANES) + kk * LANES for kk in range(DIM // LANES)]

    def fire(t, b):
        v = idx_s[t]
        off = pl.multiple_of((v >> 7) * 128, 128)
        return pltpu.async_copy(
            tokT_hbm.at[:, pl.ds(off, 128)], tcol_v.at[b], sems[b]
        )

    def process(t, b):
        v = idx_s[t]
        vl = jnp.full((LANES,), v & 127, jnp.int32)
        bsp = jnp.full((LANES,), b, jnp.int32)
        tsp = jnp.full((LANES,), t, jnp.int32)
        for kk in range(DIM // LANES):
            col = plsc.load_gather(tcol_v, [bsp, iotas[kk], vl])
            plsc.addupdate_scatter(outT_v, [iotas[kk], tsp], col)

    def drain(b):
        pltpu.make_async_copy(
            tokT_hbm.at[:, pl.ds(0, 128)], tcol_v.at[b], sems[b]
        ).wait()

    for b in range(NBUF):
        fire(b, b)

    def group(g, carry):
        for b in range(NBUF):
            t = g * NBUF + b
            drain(b)
            process(t, b)
            fire(t + NBUF, b)
        return carry

    lax.fori_loop(0, NGRP - 1, group, 0)

    for b in range(NBUF):
        t = (NGRP - 1) * NBUF + b
        drain(b)
        process(t, b)
        if b < REM:
            fire(t + NBUF, b)
    for r in range(REM):
        t = NGRP * NBUF + r
        drain(r)
        process(t, r)

    for i in range(NB_PER_W):
        pltpu.sync_copy(
            outT_v.at[:, pl.ds(i * CBLK, CBLK)],
            out_hbm.at[NB_PER_W * bb + i, :, pl.ds(c0, CBLK)],
        )


@jax.jit
def kernel(x, token_table, pos_table):
    tokT = token_table.T
    posT = pos_table.T
    mesh = plsc.VectorSubcoreMesh(core_axis_name="c", subcore_axis_name="s")
    out = pl.kernel(
        _emb_body,
        out_type=jax.ShapeDtypeStruct((BATCH, DIM, CTX), jnp.float32),
        mesh=mesh,
        scratch_types=[
            pltpu.VMEM((NB_PER_W, CBLK), jnp.int32),
            pltpu.VMEM((NBUF, DIM, 128), jnp.float32),
            pltpu.VMEM((DIM, TOK), jnp.float32),
            pltpu.SMEM((TOK,), jnp.int32),
        ] + [pltpu.SemaphoreType.DMA] * NBUF,
        compiler_params=pltpu.CompilerParams(needs_layout_passes=False),
    )(x, tokT, posT)
    return jnp.transpose(out, (0, 2, 1))


# speedup vs baseline: 7.2560x; 1.0139x over previous
import jax
import jax.numpy as jnp
from jax import lax
from jax.experimental import pallas as pl
from jax.experimental.pallas import tpu as pltpu
from jax.experimental.pallas import tpu_sc as plsc

CTX = 2048
DIM = 64
BATCH = 4

NC = 2
NS = 16
NW = NC * NS
CBLK = 128
NB_PER_W = 2
TOK = NB_PER_W * CBLK
LANES = 16
NBUF = 8
NGRP = TOK // NBUF


def _emb_body(x_hbm, tokT_hbm, posT_hbm, out_hbm, idx_v, tcol_v, outT_v,
              idx_s, *sems):
    c = lax.axis_index("c")
    s = lax.axis_index("s")
    wid = s * NC + c
    cb = wid % (CTX // CBLK)
    bb = wid // (CTX // CBLK)
    c0 = cb * CBLK

    for i in range(NB_PER_W):
        pltpu.sync_copy(
            x_hbm.at[pl.ds(NB_PER_W * bb + i, 1), pl.ds(c0, CBLK)],
            idx_v.at[pl.ds(i, 1)],
        )

    for i in range(NB_PER_W):
        pltpu.sync_copy(
            posT_hbm.at[:, pl.ds(c0, CBLK)],
            outT_v.at[:, pl.ds(i * CBLK, CBLK)],
        )

    for k in range(TOK // LANES):
        vec = idx_v[k // (CBLK // LANES), pl.ds((k % (CBLK // LANES)) * LANES, LANES)]
        for i in range(LANES):
            idx_s[k * LANES + i] = vec[i]

    iotas = [lax.iota(jnp.int32, LANES) + kk * LANES for kk in range(DIM // LANES)]

    def fire(t, b):
        v = idx_s[t]
        off = pl.multiple_of((v >> 7) * 128, 128)
        return pltpu.async_copy(
            tokT_hbm.at[:, pl.ds(off, 128)], tcol_v.at[b], sems[b]
        )

    def process(t, b):
        v = idx_s[t]
        vl = jnp.full((LANES,), v & 127, jnp.int32)
        bsp = jnp.full((LANES,), b, jnp.int32)
        tsp = jnp.full((LANES,), t, jnp.int32)
        for kk in range(DIM // LANES):
            col = plsc.load_gather(tcol_v, [bsp, iotas[kk], vl])
            plsc.addupdate_scatter(outT_v, [iotas[kk], tsp], col)

    def drain(b):
        pltpu.make_async_copy(
            tokT_hbm.at[:, pl.ds(0, 128)], tcol_v.at[b], sems[b]
        ).wait()

    for b in range(NBUF):
        fire(b, b)

    def group(g, carry):
        for b in range(NBUF):
            t = g * NBUF + b
            drain(b)
            process(t, b)
            fire(t + NBUF, b)
        return carry

    lax.fori_loop(0, NGRP - 1, group, 0)

    for b in range(NBUF):
        t = (NGRP - 1) * NBUF + b
        drain(b)
        process(t, b)

    for i in range(NB_PER_W):
        pltpu.sync_copy(
            outT_v.at[:, pl.ds(i * CBLK, CBLK)],
            out_hbm.at[NB_PER_W * bb + i, :, pl.ds(c0, CBLK)],
        )


@jax.jit
def kernel(x, token_table, pos_table):
    tokT = token_table.T
    posT = pos_table.T
    mesh = plsc.VectorSubcoreMesh(core_axis_name="c", subcore_axis_name="s")
    out = pl.kernel(
        _emb_body,
        out_type=jax.ShapeDtypeStruct((BATCH, DIM, CTX), jnp.float32),
        mesh=mesh,
        scratch_types=[
            pltpu.VMEM((NB_PER_W, CBLK), jnp.int32),
            pltpu.VMEM((NBUF, DIM, 128), jnp.float32),
            pltpu.VMEM((DIM, TOK), jnp.float32),
            pltpu.SMEM((TOK,), jnp.int32),
        ] + [pltpu.SemaphoreType.DMA] * NBUF,
        compiler_params=pltpu.CompilerParams(needs_layout_passes=False, disable_bounds_checks=True),
    )(x, tokT, posT)
    return jnp.transpose(out, (0, 2, 1))


# speedup vs baseline: 7.2725x; 1.0023x over previous
import jax
import jax.numpy as jnp
from jax import lax
from jax.experimental import pallas as pl
from jax.experimental.pallas import tpu as pltpu
from jax.experimental.pallas import tpu_sc as plsc

CTX = 2048
DIM = 64
BATCH = 4

NC = 2
NS = 16
NW = NC * NS
CBLK = 128
NB_PER_W = 2
TOK = NB_PER_W * CBLK
LANES = 16
NBUF = 8
NGRP = TOK // NBUF


def _emb_body(x_hbm, tokT_hbm, posT_hbm, out_hbm, idx_v, tcol_v, outT_v,
              idx_s, *sems):
    c = lax.axis_index("c")
    s = lax.axis_index("s")
    wid = s * NC + c
    cb = wid % (CTX // CBLK)
    bb = wid // (CTX // CBLK)
    c0 = cb * CBLK

    for i in range(NB_PER_W):
        pltpu.sync_copy(
            x_hbm.at[pl.ds(NB_PER_W * bb + i, 1), pl.ds(c0, CBLK)],
            idx_v.at[pl.ds(i, 1)],
        )

    for i in range(NB_PER_W):
        pltpu.sync_copy(
            posT_hbm.at[:, pl.ds(c0, CBLK)],
            outT_v.at[:, pl.ds(i * CBLK, CBLK)],
        )

    for k in range(TOK // LANES):
        vec = idx_v[k // (CBLK // LANES), pl.ds((k % (CBLK // LANES)) * LANES, LANES)]
        for i in range(LANES):
            idx_s[k * LANES + i] = vec[i]

    iotas = [lax.iota(jnp.int32, LANES) + kk * LANES for kk in range(DIM // LANES)]

    def fire(t, b):
        v = idx_s[t]
        off = pl.multiple_of((v >> 7) * 128, 128)
        return pltpu.async_copy(
            tokT_hbm.at[:, pl.ds(off, 128)], tcol_v.at[b], sems[b]
        )

    def process(t, b):
        v = idx_s[t]
        vl = jnp.full((LANES,), v & 127, jnp.int32)
        bsp = jnp.full((LANES,), b, jnp.int32)
        tsp = jnp.full((LANES,), t, jnp.int32)
        for kk in range(DIM // LANES):
            col = plsc.load_gather(tcol_v, [bsp, iotas[kk], vl])
            plsc.addupdate_scatter(outT_v, [iotas[kk], tsp], col)

    def drain(b):
        pltpu.make_async_copy(
            tokT_hbm.at[:, pl.ds(0, 128)], tcol_v.at[b], sems[b]
        ).wait()

    for b in range(NBUF):
        fire(b, b)

    def group(g, carry):
        for b in range(NBUF):
            t = g * NBUF + b
            drain(b)
            process(t, b)
            fire(t + NBUF, b)
        return carry

    lax.fori_loop(0, NGRP - 1, group, 0)

    for b in range(NBUF):
        t = (NGRP - 1) * NBUF + b
        drain(b)
        process(t, b)

    for i in range(NB_PER_W):
        pltpu.sync_copy(
            outT_v.at[:, pl.ds(i * CBLK, CBLK)],
            out_hbm.at[NB_PER_W * bb + i, :, pl.ds(c0, CBLK)],
        )


@jax.jit
def kernel(x, token_table, pos_table):
    tokT = token_table.T
    posT = pos_table.T
    mesh = plsc.VectorSubcoreMesh(core_axis_name="c", subcore_axis_name="s")
    out = pl.kernel(
        _emb_body,
        out_type=jax.ShapeDtypeStruct((BATCH, DIM, CTX), jnp.float32),
        mesh=mesh,
        scratch_types=[
            pltpu.VMEM((NB_PER_W, CBLK), jnp.int32),
            pltpu.VMEM((NBUF, DIM, 128), jnp.float32),
            pltpu.VMEM((DIM, TOK), jnp.float32),
            pltpu.SMEM((TOK,), jnp.int32),
        ] + [pltpu.SemaphoreType.DMA] * NBUF,
        compiler_params=pltpu.CompilerParams(needs_layout_passes=False),
    )(x, tokT, posT)
    return jnp.transpose(out, (0, 2, 1))


# speedup vs baseline: 7.3454x; 1.0100x over previous
import jax
import jax.numpy as jnp
from jax import lax
from jax.experimental import pallas as pl
from jax.experimental.pallas import tpu as pltpu
from jax.experimental.pallas import tpu_sc as plsc

CTX = 2048
DIM = 64
BATCH = 4

NC = 2
NS = 16
NW = NC * NS
CBLK = 128
NB_PER_W = 2
TOK = NB_PER_W * CBLK
LANES = 16
NBUF = 8
NGRP = TOK // NBUF


def _emb_body(x_hbm, tokT_hbm, posT_hbm, out_hbm, idx_v, tcol_v, outT_v,
              idx_s, *sems):
    c = lax.axis_index("c")
    s = lax.axis_index("s")
    wid = s * NC + c
    cb = wid % (CTX // CBLK)
    bb = wid // (CTX // CBLK)
    c0 = cb * CBLK

    for i in range(NB_PER_W):
        pltpu.sync_copy(
            x_hbm.at[pl.ds(NB_PER_W * bb + i, 1), pl.ds(c0, CBLK)],
            idx_v.at[pl.ds(i, 1)],
        )

    pos_copies = [
        pltpu.async_copy(
            posT_hbm.at[:, pl.ds(c0, CBLK)],
            outT_v.at[:, pl.ds(i * CBLK, CBLK)],
            sems[NBUF],
        )
        for i in range(NB_PER_W)
    ]

    for k in range(TOK // LANES):
        vec = idx_v[k // (CBLK // LANES), pl.ds((k % (CBLK // LANES)) * LANES, LANES)]
        for i in range(LANES):
            idx_s[k * LANES + i] = vec[i]

    iotas = [lax.iota(jnp.int32, LANES) + kk * LANES for kk in range(DIM // LANES)]

    def fire(t, b):
        v = idx_s[t]
        off = pl.multiple_of((v >> 7) * 128, 128)
        return pltpu.async_copy(
            tokT_hbm.at[:, pl.ds(off, 128)], tcol_v.at[b], sems[b]
        )

    def process(t, b):
        v = idx_s[t]
        vl = jnp.full((LANES,), v & 127, jnp.int32)
        bsp = jnp.full((LANES,), b, jnp.int32)
        tsp = jnp.full((LANES,), t, jnp.int32)
        for kk in range(DIM // LANES):
            col = plsc.load_gather(tcol_v, [bsp, iotas[kk], vl])
            plsc.addupdate_scatter(outT_v, [iotas[kk], tsp], col)

    def drain(b):
        pltpu.make_async_copy(
            tokT_hbm.at[:, pl.ds(0, 128)], tcol_v.at[b], sems[b]
        ).wait()

    for b in range(NBUF):
        fire(b, b)
    for cp in pos_copies:
        cp.wait()

    def group(g, carry):
        for b in range(NBUF):
            t = g * NBUF + b
            drain(b)
            process(t, b)
            fire(t + NBUF, b)
        return carry

    lax.fori_loop(0, NGRP - 1, group, 0)

    for b in range(NBUF):
        t = (NGRP - 1) * NBUF + b
        drain(b)
        process(t, b)

    wb_copies = [
        pltpu.async_copy(
            outT_v.at[:, pl.ds(i * CBLK, CBLK)],
            out_hbm.at[NB_PER_W * bb + i, :, pl.ds(c0, CBLK)],
            sems[NBUF],
        )
        for i in range(NB_PER_W)
    ]
    for cp in wb_copies:
        cp.wait()


@jax.jit
def kernel(x, token_table, pos_table):
    tokT = token_table.T
    posT = pos_table.T
    mesh = plsc.VectorSubcoreMesh(core_axis_name="c", subcore_axis_name="s")
    out = pl.kernel(
        _emb_body,
        out_type=jax.ShapeDtypeStruct((BATCH, DIM, CTX), jnp.float32),
        mesh=mesh,
        scratch_types=[
            pltpu.VMEM((NB_PER_W, CBLK), jnp.int32),
            pltpu.VMEM((NBUF, DIM, 128), jnp.float32),
            pltpu.VMEM((DIM, TOK), jnp.float32),
            pltpu.SMEM((TOK,), jnp.int32),
        ] + [pltpu.SemaphoreType.DMA] * (NBUF + 1),
        compiler_params=pltpu.CompilerParams(needs_layout_passes=False),
    )(x, tokT, posT)
    return jnp.transpose(out, (0, 2, 1))
